# Initial kernel scaffold; baseline (speedup 1.0000x reference)
import jax
import jax.numpy as jnp
from jax.experimental import pallas as pl


def kernel(x, ei, W1, b1, W2, b2, W3, b3):
    raise NotImplementedError("write your pallas kernel here")


# speedup vs baseline: 10.4926x; 10.4926x over previous
import functools

import jax
import jax.numpy as jnp
from jax import lax
from jax.experimental import pallas as pl
from jax.experimental.pallas import tpu as pltpu
from jax.experimental.pallas import tpu_sc as plsc

N = 10000
E = 320000
IN_DIM = 128
HIDDEN = 128
OUT = 64

NC = 2
NS = 16
K = 128
CH = 79
E_PAD = NC * NS * CH * K
N_PAD = 10112
RPT = N_PAD // NS

_MESH = plsc.VectorSubcoreMesh(core_axis_name="c", subcore_axis_name="s")
_ROW_CHUNKS = [(0, K), (K, K), (2 * K, K), (3 * K, K), (4 * K, RPT - 4 * K)]


@functools.partial(
    pl.kernel,
    out_type=jax.ShapeDtypeStruct((NC * N_PAD,), jnp.float32),
    mesh=_MESH,
    scratch_types=[
        pltpu.VMEM_SHARED((N_PAD,), jnp.float32),
        pltpu.VMEM((CH, K), jnp.int32),
        pltpu.VMEM((K,), jnp.float32),
        pltpu.VMEM((RPT,), jnp.float32),
    ],
)
def _degree_kernel(dst_hbm, init_hbm, deg_out, deg_sh, idx_v, ones_v, stage_v):
    c = lax.axis_index("c")
    s = lax.axis_index("s")
    pltpu.sync_copy(init_hbm.at[pl.ds(c * N_PAD + s * RPT, RPT)], stage_v)
    pltpu.sync_copy(stage_v, deg_sh.at[pl.ds(s * RPT, RPT)])
    pltpu.sync_copy(dst_hbm.at[c, s], idx_v)
    for i in range(K // 16):
        ones_v[pl.ds(i * 16, 16)] = jnp.ones((16,), jnp.float32)
    plsc.subcore_barrier()

    def body(j, carry):
        pltpu.sync_copy(ones_v, deg_sh.at[idx_v.at[j]], add=True)
        return carry

    lax.fori_loop(0, CH, body, 0)
    plsc.subcore_barrier()
    pltpu.sync_copy(deg_sh.at[pl.ds(s * RPT, RPT)], stage_v)
    pltpu.sync_copy(stage_v, deg_out.at[pl.ds(c * N_PAD + s * RPT, RPT)])


def _make_propagate(d):
    @functools.partial(
        pl.kernel,
        out_type=jax.ShapeDtypeStruct((NC, N_PAD, d), jnp.float32),
        mesh=_MESH,
        scratch_types=[
            pltpu.VMEM_SHARED((N_PAD, d), jnp.float32),
            pltpu.VMEM((CH, K), jnp.int32),
            pltpu.VMEM((CH, K), jnp.int32),
            pltpu.VMEM((K, d), jnp.float32),
        ],
    )
    def propagate(p_hbm, src_hbm, dst_hbm, out_hbm, acc_sh, src_v, dst_v,
                  rows_v):
        c = lax.axis_index("c")
        s = lax.axis_index("s")
        for off, sz in _ROW_CHUNKS:
            pltpu.sync_copy(p_hbm.at[pl.ds(s * RPT + off, sz)],
                            rows_v.at[pl.ds(0, sz)])
            pltpu.sync_copy(rows_v.at[pl.ds(0, sz)],
                            acc_sh.at[pl.ds(s * RPT + off, sz)])
        pltpu.sync_copy(src_hbm.at[c, s], src_v)
        pltpu.sync_copy(dst_hbm.at[c, s], dst_v)
        plsc.subcore_barrier()

        def body(j, carry):
            pltpu.sync_copy(p_hbm.at[src_v.at[j]], rows_v)
            pltpu.sync_copy(rows_v, acc_sh.at[dst_v.at[j]], add=True)
            return carry

        lax.fori_loop(0, CH, body, 0)
        plsc.subcore_barrier()
        for off, sz in _ROW_CHUNKS:
            pltpu.sync_copy(acc_sh.at[pl.ds(s * RPT + off, sz)],
                            rows_v.at[pl.ds(0, sz)])
            pltpu.sync_copy(rows_v.at[pl.ds(0, sz)],
                            out_hbm.at[c, pl.ds(s * RPT + off, sz)])

    return propagate


_propagate_h = _make_propagate(HIDDEN)


def _entry_body(degp_ref, x_ref, w_ref, dinv_ref, p_ref):
    deg = degp_ref[0, :] + degp_ref[1, :]
    dinv = lax.rsqrt(deg)
    dinv_ref[...] = dinv[:, None]
    p_ref[...] = dinv[:, None] * jnp.dot(
        x_ref[...], w_ref[...], preferred_element_type=jnp.float32)


_tc_entry = pl.pallas_call(
    _entry_body,
    out_shape=(
        jax.ShapeDtypeStruct((N_PAD, 1), jnp.float32),
        jax.ShapeDtypeStruct((N_PAD, IN_DIM), jnp.float32),
    ),
)


def _mid_body(s_ref, p_ref, dinv_ref, b_ref, w_ref, o_ref):
    tot = s_ref[0] + s_ref[1] - p_ref[...]
    h = jnp.maximum(dinv_ref[...] * tot + b_ref[...], 0.0)
    o_ref[...] = dinv_ref[...] * jnp.dot(
        h, w_ref[...], preferred_element_type=jnp.float32)


def _make_mid(d_out):
    return pl.pallas_call(
        _mid_body,
        out_shape=jax.ShapeDtypeStruct((N_PAD, d_out), jnp.float32),
    )


_tc_mid2 = _make_mid(HIDDEN)
_tc_mid3 = _make_mid(HIDDEN)


def _final_body(s_ref, p_ref, dinv_ref, b_ref, o_ref):
    tot = s_ref[0, :, :OUT] + s_ref[1, :, :OUT] - p_ref[:, :OUT]
    o_ref[...] = dinv_ref[...] * tot + b_ref[...]


_tc_final = pl.pallas_call(
    _final_body,
    out_shape=jax.ShapeDtypeStruct((N_PAD, OUT), jnp.float32),
)


def kernel(x, ei, W1, b1, W2, b2, W3, b3):
    pad = jnp.full((E_PAD - E,), N, dtype=jnp.int32)
    src_t = jnp.concatenate([ei[0].astype(jnp.int32), pad]).reshape(NC, NS, CH, K)
    dst_t = jnp.concatenate([ei[1].astype(jnp.int32), pad]).reshape(NC, NS, CH, K)
    xp = jnp.pad(x, ((0, N_PAD - N), (0, 0)))
    init = jnp.concatenate([jnp.ones((N_PAD,), jnp.float32),
                            jnp.zeros((N_PAD,), jnp.float32)])

    degp = _degree_kernel(dst_t, init).reshape(NC, N_PAD)
    dinv, p1 = _tc_entry(degp, xp, W1)
    s1 = _propagate_h(p1, src_t, dst_t)
    p2 = _tc_mid2(s1, p1, dinv, b1.reshape(1, -1), W2)
    s2 = _propagate_h(p2, src_t, dst_t)
    w3p = jnp.pad(W3, ((0, 0), (0, HIDDEN - OUT)))
    p3 = _tc_mid3(s2, p2, dinv, b2.reshape(1, -1), w3p)
    s3 = _propagate_h(p3, src_t, dst_t)
    out = _tc_final(s3, p3, dinv, b3.reshape(1, -1))
    return out[:N]
